# baseline (speedup 1.0000x reference)
import functools

import jax
import jax.numpy as jnp
from jax import lax
from jax.experimental import pallas as pl
from jax.experimental.pallas import tpu as pltpu
from jax.experimental.pallas import tpu_sc as plsc

EMB_DIM = 128
B_POS = 16384
B_NEG = 81920

NC = 2
NS = 16
NW = NC * NS
PPW = B_POS // NW
PNW = B_NEG // NW
PER_W = PPW + PNW
CHUNK = 128
NCHUNK = PER_W // CHUNK
POS_VECS = PPW // 16
DEPTH = 3

_mesh = plsc.VectorSubcoreMesh(core_axis_name="c", subcore_axis_name="s")


@functools.partial(
    pl.kernel,
    mesh=_mesh,
    out_type=jax.ShapeDtypeStruct((NW * 16,), jnp.float32),
    scratch_types=[
        pltpu.VMEM((PER_W,), jnp.int32),
        pltpu.VMEM((PER_W,), jnp.int32),
        pltpu.VMEM((CHUNK, EMB_DIM), jnp.float32),
        pltpu.VMEM((CHUNK, EMB_DIM), jnp.float32),
        pltpu.VMEM((CHUNK, EMB_DIM), jnp.float32),
        pltpu.VMEM((CHUNK, EMB_DIM), jnp.float32),
        pltpu.VMEM((CHUNK, EMB_DIM), jnp.float32),
        pltpu.VMEM((CHUNK, EMB_DIM), jnp.float32),
        pltpu.VMEM((256,), jnp.float32),
        pltpu.VMEM((256,), jnp.float32),
        pltpu.VMEM((PER_W,), jnp.float32),
        pltpu.VMEM((16,), jnp.float32),
        pltpu.SemaphoreType.DMA,
        pltpu.SemaphoreType.DMA,
        pltpu.SemaphoreType.DMA,
        pltpu.SemaphoreType.DMA,
        pltpu.SemaphoreType.DMA,
        pltpu.SemaphoreType.DMA,
    ],
    compiler_params=pltpu.CompilerParams(needs_layout_passes=False),
)
def _sc_loss_parts(pos_w_hbm, pos_v_hbm, neg_w_hbm, neg_v_hbm,
                   w_hbm, v_hbm, out_hbm,
                   idxw_v, idxv_v, wrows0, vrows0, wrows1, vrows1,
                   wrows2, vrows2, pbuf_a, pbuf_b, sc_v, part_v,
                   sem_w0, sem_v0, sem_w1, sem_v1, sem_w2, sem_v2):
    wid = lax.axis_index("s") * NC + lax.axis_index("c")
    lane = lax.broadcasted_iota(jnp.int32, (16,), 0)

    bufs = ((wrows0, vrows0, sem_w0, sem_v0),
            (wrows1, vrows1, sem_w1, sem_v1),
            (wrows2, vrows2, sem_w2, sem_v2))

    def gather_start(c, slot):
        w_r, v_r, s_w, s_v = bufs[slot]
        pltpu.async_copy(w_hbm.at[idxw_v.at[pl.ds(c * CHUNK, CHUNK)]],
                         w_r, s_w)
        pltpu.async_copy(v_hbm.at[idxv_v.at[pl.ds(c * CHUNK, CHUNK)]],
                         v_r, s_v)

    def gather_wait(c, slot):
        w_r, v_r, s_w, s_v = bufs[slot]
        pltpu.make_async_copy(
            w_hbm.at[idxw_v.at[pl.ds(c * CHUNK, CHUNK)]], w_r, s_w).wait()
        pltpu.make_async_copy(
            v_hbm.at[idxv_v.at[pl.ds(c * CHUNK, CHUNK)]], v_r, s_v).wait()

    def dots_16(wrows, vrows, g, pbuf):
        def lds(i):
            p = g * 16 + i
            return ([wrows[p, pl.ds(j * 16, 16)] for j in range(8)],
                    [vrows[p, pl.ds(j * 16, 16)] for j in range(8)])

        wv, vv = lds(0)
        for i in range(16):
            nxt = lds(i + 1) if i < 15 else None
            t = [wv[j] * vv[j] for j in range(8)]
            s0 = t[0] + t[1]
            s1 = t[2] + t[3]
            s2 = t[4] + t[5]
            s3 = t[6] + t[7]
            pbuf[pl.ds(i * 16, 16)] = (s0 + s1) + (s2 + s3)
            if nxt is not None:
                wv, vv = nxt
        acc0 = plsc.load_gather(pbuf, [lane * 16])
        acc1 = plsc.load_gather(pbuf, [lane * 16 + 1])
        for j in range(2, 16, 2):
            acc0 = acc0 + plsc.load_gather(pbuf, [lane * 16 + j])
            acc1 = acc1 + plsc.load_gather(pbuf, [lane * 16 + j + 1])
        return acc0 + acc1

    def compute(c, slot):
        wrows, vrows, _, _ = bufs[slot]

        def group_body(g2, carry2):
            ga = g2 * 2
            sc_v[pl.ds(c * CHUNK + ga * 16, 16)] = (
                dots_16(wrows, vrows, ga, pbuf_a))
            sc_v[pl.ds(c * CHUNK + (ga + 1) * 16, 16)] = (
                dots_16(wrows, vrows, ga + 1, pbuf_b))
            return carry2

        lax.fori_loop(0, CHUNK // 32, group_body, 0)

    cp0 = pltpu.async_copy(pos_w_hbm.at[pl.ds(wid * PPW, CHUNK)],
                           idxw_v.at[pl.ds(0, CHUNK)], sem_w2)
    cp1 = pltpu.async_copy(pos_v_hbm.at[pl.ds(wid * PPW, CHUNK)],
                           idxv_v.at[pl.ds(0, CHUNK)], sem_v2)
    cp0.wait()
    cp1.wait()
    gather_start(0, 0)
    idx_copies = (
        pltpu.async_copy(pos_w_hbm.at[pl.ds(wid * PPW + CHUNK, PPW - CHUNK)],
                         idxw_v.at[pl.ds(CHUNK, PPW - CHUNK)], sem_w1),
        pltpu.async_copy(neg_w_hbm.at[pl.ds(wid * PNW, PNW)],
                         idxw_v.at[pl.ds(PPW, PNW)], sem_v1),
        pltpu.async_copy(pos_v_hbm.at[pl.ds(wid * PPW + CHUNK, PPW - CHUNK)],
                         idxv_v.at[pl.ds(CHUNK, PPW - CHUNK)], sem_w2),
        pltpu.async_copy(neg_v_hbm.at[pl.ds(wid * PNW, PNW)],
                         idxv_v.at[pl.ds(PPW, PNW)], sem_v2),
    )
    for cp in idx_copies:
        cp.wait()
    for s in range(1, DEPTH):
        gather_start(s, s)

    def pipe_body(cc, carry):
        c = cc * DEPTH
        for s in range(DEPTH):
            gather_wait(c + s, s)
            compute(c + s, s)

            @pl.when(c + s + DEPTH < NCHUNK)
            def _():
                gather_start(c + s + DEPTH, s)
        return carry

    lax.fori_loop(0, NCHUNK // DEPTH, pipe_body, 0)

    def ls_one(k):
        s = sc_v[pl.ds(k * 16, 16)]
        t = jnp.where(k < POS_VECS, -s, s)
        u = jnp.exp(-jnp.abs(t))
        r = u / (2.0 + u)
        r2 = r * r
        l1p = 2.0 * r * (1.0 + r2 * (1.0 / 3.0 + r2 * (0.2 + r2 / 7.0)))
        return jnp.minimum(t, 0.0) - l1p

    def ls_body(k4, accs):
        a0, a1, a2, a3 = accs
        k = k4 * 4
        return (a0 + ls_one(k), a1 + ls_one(k + 1),
                a2 + ls_one(k + 2), a3 + ls_one(k + 3))

    zero = jnp.zeros((16,), jnp.float32)
    p0, p1, p2, p3 = lax.fori_loop(0, PER_W // 64, ls_body,
                                   (zero, zero, zero, zero))
    part_v[...] = (p0 + p1) + (p2 + p3)
    pltpu.sync_copy(part_v, out_hbm.at[pl.ds(wid * 16, 16)])


def _loss_body(s_ref, o_ref):
    o_ref[0, 0] = -jnp.sum(s_ref[...])


def kernel(pos_w, pos_v, neg_w, neg_v, w_emb, v_emb):
    parts = _sc_loss_parts(pos_w.astype(jnp.int32), pos_v.astype(jnp.int32),
                           neg_w.astype(jnp.int32), neg_v.astype(jnp.int32),
                           w_emb, v_emb)
    loss = pl.pallas_call(
        _loss_body,
        out_shape=jax.ShapeDtypeStruct((1, 1), jnp.float32),
        out_specs=pl.BlockSpec(memory_space=pltpu.SMEM),
    )(parts.reshape(NW * 16 // 128, 128))
    return loss[0, 0]


# speedup vs baseline: 1.0219x; 1.0219x over previous
import functools

import jax
import jax.numpy as jnp
from jax import lax
from jax.experimental import pallas as pl
from jax.experimental.pallas import tpu as pltpu
from jax.experimental.pallas import tpu_sc as plsc

EMB_DIM = 128
B_POS = 16384
B_NEG = 81920

NC = 2
NS = 16
NW = NC * NS
PPW = B_POS // NW
PNW = B_NEG // NW
PER_W = PPW + PNW
CHUNK = 128
NCHUNK = PER_W // CHUNK
POS_VECS = PPW // 16
DEPTH = 3

_mesh = plsc.VectorSubcoreMesh(core_axis_name="c", subcore_axis_name="s")


@functools.partial(
    pl.kernel,
    mesh=_mesh,
    out_type=jax.ShapeDtypeStruct((NW * 16,), jnp.float32),
    scratch_types=[
        pltpu.VMEM((PER_W,), jnp.int32),
        pltpu.VMEM((PER_W,), jnp.int32),
        pltpu.VMEM((CHUNK, EMB_DIM), jnp.float32),
        pltpu.VMEM((CHUNK, EMB_DIM), jnp.float32),
        pltpu.VMEM((CHUNK, EMB_DIM), jnp.float32),
        pltpu.VMEM((CHUNK, EMB_DIM), jnp.float32),
        pltpu.VMEM((CHUNK, EMB_DIM), jnp.float32),
        pltpu.VMEM((CHUNK, EMB_DIM), jnp.float32),
        pltpu.VMEM((256,), jnp.float32),
        pltpu.VMEM((256,), jnp.float32),
        pltpu.VMEM((PER_W,), jnp.float32),
        pltpu.VMEM((16,), jnp.float32),
        pltpu.SemaphoreType.DMA,
        pltpu.SemaphoreType.DMA,
        pltpu.SemaphoreType.DMA,
        pltpu.SemaphoreType.DMA,
        pltpu.SemaphoreType.DMA,
        pltpu.SemaphoreType.DMA,
    ],
    compiler_params=pltpu.CompilerParams(needs_layout_passes=False),
)
def _sc_loss_parts(pos_w_hbm, pos_v_hbm, neg_w_hbm, neg_v_hbm,
                   w_hbm, v_hbm, out_hbm,
                   idxw_v, idxv_v, wrows0, vrows0, wrows1, vrows1,
                   wrows2, vrows2, pbuf_a, pbuf_b, sc_v, part_v,
                   sem_w0, sem_v0, sem_w1, sem_v1, sem_w2, sem_v2):
    wid = lax.axis_index("s") * NC + lax.axis_index("c")
    lane = lax.broadcasted_iota(jnp.int32, (16,), 0)

    idx_copies = (
        pltpu.async_copy(pos_w_hbm.at[pl.ds(wid * PPW, PPW)],
                         idxw_v.at[pl.ds(0, PPW)], sem_w0),
        pltpu.async_copy(neg_w_hbm.at[pl.ds(wid * PNW, PNW)],
                         idxw_v.at[pl.ds(PPW, PNW)], sem_v0),
        pltpu.async_copy(pos_v_hbm.at[pl.ds(wid * PPW, PPW)],
                         idxv_v.at[pl.ds(0, PPW)], sem_w1),
        pltpu.async_copy(neg_v_hbm.at[pl.ds(wid * PNW, PNW)],
                         idxv_v.at[pl.ds(PPW, PNW)], sem_v1),
    )
    for cp in idx_copies:
        cp.wait()

    bufs = ((wrows0, vrows0, sem_w0, sem_v0),
            (wrows1, vrows1, sem_w1, sem_v1),
            (wrows2, vrows2, sem_w2, sem_v2))

    def gather_start(c, slot):
        w_r, v_r, s_w, s_v = bufs[slot]
        pltpu.async_copy(w_hbm.at[idxw_v.at[pl.ds(c * CHUNK, CHUNK)]],
                         w_r, s_w)
        pltpu.async_copy(v_hbm.at[idxv_v.at[pl.ds(c * CHUNK, CHUNK)]],
                         v_r, s_v)

    def gather_wait(c, slot):
        w_r, v_r, s_w, s_v = bufs[slot]
        pltpu.make_async_copy(
            w_hbm.at[idxw_v.at[pl.ds(c * CHUNK, CHUNK)]], w_r, s_w).wait()
        pltpu.make_async_copy(
            v_hbm.at[idxv_v.at[pl.ds(c * CHUNK, CHUNK)]], v_r, s_v).wait()

    def dots_16(wrows, vrows, g, pbuf):
        def lds(i):
            p = g * 16 + i
            return ([wrows[p, pl.ds(j * 16, 16)] for j in range(8)],
                    [vrows[p, pl.ds(j * 16, 16)] for j in range(8)])

        wv, vv = lds(0)
        for i in range(16):
            nxt = lds(i + 1) if i < 15 else None
            t = [wv[j] * vv[j] for j in range(8)]
            s0 = t[0] + t[1]
            s1 = t[2] + t[3]
            s2 = t[4] + t[5]
            s3 = t[6] + t[7]
            pbuf[pl.ds(i * 16, 16)] = (s0 + s1) + (s2 + s3)
            if nxt is not None:
                wv, vv = nxt
        acc0 = plsc.load_gather(pbuf, [lane * 16])
        acc1 = plsc.load_gather(pbuf, [lane * 16 + 1])
        for j in range(2, 16, 2):
            acc0 = acc0 + plsc.load_gather(pbuf, [lane * 16 + j])
            acc1 = acc1 + plsc.load_gather(pbuf, [lane * 16 + j + 1])
        return acc0 + acc1

    def compute(c, slot):
        wrows, vrows, _, _ = bufs[slot]

        def group_body(g2, carry2):
            ga = g2 * 2
            sc_v[pl.ds(c * CHUNK + ga * 16, 16)] = (
                dots_16(wrows, vrows, ga, pbuf_a))
            sc_v[pl.ds(c * CHUNK + (ga + 1) * 16, 16)] = (
                dots_16(wrows, vrows, ga + 1, pbuf_b))
            return carry2

        lax.fori_loop(0, CHUNK // 32, group_body, 0)

    for s in range(DEPTH):
        gather_start(s, s)

    def pipe_body(cc, carry):
        c = cc * DEPTH
        for s in range(DEPTH):
            gather_wait(c + s, s)
            compute(c + s, s)

            @pl.when(c + s + DEPTH < NCHUNK)
            def _():
                gather_start(c + s + DEPTH, s)
        return carry

    lax.fori_loop(0, NCHUNK // DEPTH, pipe_body, 0)

    def ls_one(k):
        s = sc_v[pl.ds(k * 16, 16)]
        t = jnp.where(k < POS_VECS, -s, s)
        u = jnp.exp(-jnp.abs(t))
        r = u / (2.0 + u)
        r2 = r * r
        l1p = 2.0 * r * (1.0 + r2 * (1.0 / 3.0 + r2 * (0.2 + r2 / 7.0)))
        return jnp.minimum(t, 0.0) - l1p

    def ls_body(k4, accs):
        a0, a1, a2, a3 = accs
        k = k4 * 4
        return (a0 + ls_one(k), a1 + ls_one(k + 1),
                a2 + ls_one(k + 2), a3 + ls_one(k + 3))

    zero = jnp.zeros((16,), jnp.float32)
    p0, p1, p2, p3 = lax.fori_loop(0, PER_W // 64, ls_body,
                                   (zero, zero, zero, zero))
    part_v[...] = (p0 + p1) + (p2 + p3)
    pltpu.sync_copy(part_v, out_hbm.at[pl.ds(wid * 16, 16)])


def _loss_body(s_ref, o_ref):
    o_ref[0, 0] = -jnp.sum(s_ref[...])


def kernel(pos_w, pos_v, neg_w, neg_v, w_emb, v_emb):
    parts = _sc_loss_parts(pos_w.astype(jnp.int32), pos_v.astype(jnp.int32),
                           neg_w.astype(jnp.int32), neg_v.astype(jnp.int32),
                           w_emb, v_emb)
    loss = pl.pallas_call(
        _loss_body,
        out_shape=jax.ShapeDtypeStruct((1, 1), jnp.float32),
        out_specs=pl.BlockSpec(memory_space=pltpu.SMEM),
    )(parts.reshape(NW * 16 // 128, 128))
    return loss[0, 0]
